# baseline (speedup 1.0000x reference)
import numpy as np
import jax
import jax.numpy as jnp
from jax import lax
from jax.experimental import pallas as pl
from jax.experimental.pallas import tpu as pltpu
from jax.experimental.pallas import tpu_sc as plsc

_HASH_PRIMES = [2654435761, 2246822519, 3266489917, 2028178513, 1220703125,
                1610612741, 805306457, 402653189, 3674653429, 2860486313,
                1073676287, 2971215073, 1500450271, 3267000013, 2654435789,
                4049292737, 2246822531, 3266489927, 2028178519, 1220703133]

_B, _T = 16, 2048
_STATE = 256
_HEADS = 4
_BITS = 16
_BUCKETS = 131072
_KEYBITS = 17
_EMB = 32
_N = _B * _T
_NROWS = _N * _HEADS
_HPAD = 8

_PBIG = np.zeros((_HEADS * _BITS, 128), np.float32)
_POWT = np.zeros((_HPAD, 128), np.float32)
for _h in range(_HEADS):
    for _i in range(_BITS):
        _p = _HASH_PRIMES[(_h * 3 + _i) % len(_HASH_PRIMES)] & (_BUCKETS - 1)
        for _j in range(_KEYBITS):
            if (_p >> _j) & 1:
                _PBIG[_h * _BITS + _i, _h * _KEYBITS + _j] = 1.0
    for _j in range(_KEYBITS):
        _POWT[_h, _h * _KEYBITS + _j] = float(1 << _j)
_OFFT = np.zeros((_HPAD, 128), np.float32)
for _h in range(_HEADS):
    _OFFT[_h, :] = float(_h * _BUCKETS)

_BLK = 2048


def _Z():
    return jnp.int32(0)


def _keys_body(x_ref, wp_ref, bp_ref, pb_ref, pwt_ref, offt_ref, out_ref):
    x = x_ref[...]
    p = jnp.dot(x, wp_ref[...], preferred_element_type=jnp.float32)
    p = p + bp_ref[...]
    bits = (p > 0).astype(jnp.float32)
    counts = jnp.dot(bits, pb_ref[...],
                     preferred_element_type=jnp.float32)
    par = counts - 2.0 * jnp.floor(counts * 0.5)
    par_t = jnp.transpose(par, (1, 0))
    key_t = jnp.dot(pwt_ref[...], par_t,
                    preferred_element_type=jnp.float32)
    idx_t = key_t + offt_ref[...][:, :1]
    out_ref[...] = idx_t.astype(jnp.int32)


def _compute_keys(x, Wp, bp):
    grid = _N // _BLK
    return pl.pallas_call(
        _keys_body,
        grid=(grid,),
        in_specs=[
            pl.BlockSpec((_BLK, _STATE), lambda i: (i, _Z())),
            pl.BlockSpec((_STATE, _HEADS * _BITS), lambda i: (_Z(), _Z())),
            pl.BlockSpec((1, _HEADS * _BITS), lambda i: (_Z(), _Z())),
            pl.BlockSpec((_HEADS * _BITS, 128), lambda i: (_Z(), _Z())),
            pl.BlockSpec((_HPAD, 128), lambda i: (_Z(), _Z())),
            pl.BlockSpec((_HPAD, 128), lambda i: (_Z(), _Z())),
        ],
        out_specs=pl.BlockSpec((_HPAD, _BLK), lambda i: (_Z(), i)),
        out_shape=jax.ShapeDtypeStruct((_HPAD, _N), jnp.int32),
    )(x, Wp, bp.reshape(1, -1), jnp.asarray(_PBIG), jnp.asarray(_POWT),
      jnp.asarray(_OFFT))


_GATHER_W = 128
_NW = 32
_TOK_W = _N // _NW
_J_W = _TOK_W // _GATHER_W


def _gather_body(table_hbm, idx_hbm, out_hbm, idx_v, buf_v, sem):
    wid = lax.axis_index("s") * 2 + lax.axis_index("c")
    tbase = wid * _TOK_W
    for h in range(_HEADS):
        pltpu.sync_copy(idx_hbm.at[jnp.int32(h), pl.ds(tbase, _TOK_W)],
                        idx_v.at[pl.ds(h * _TOK_W, _TOK_W)])

    for h in range(_HEADS):
        def step(j, carry, h=h):
            idx_row = idx_v.at[pl.ds(h * _TOK_W + j * _GATHER_W, _GATHER_W)]
            pltpu.async_copy(table_hbm.at[idx_row], buf_v, sem).wait()
            pltpu.sync_copy(
                buf_v,
                out_hbm.at[jnp.int32(h), pl.ds(tbase + j * _GATHER_W, _GATHER_W)])
            return carry

        lax.fori_loop(jnp.int32(0), jnp.int32(_J_W), step, jnp.int32(0))


def _gather(flat_table, idx_t):
    run = pl.kernel(
        _gather_body,
        mesh=plsc.VectorSubcoreMesh(core_axis_name="c", subcore_axis_name="s"),
        out_type=jax.ShapeDtypeStruct((_HEADS, _N, _EMB), jnp.float32),
        scratch_types=[
            pltpu.VMEM((_HEADS * _TOK_W,), jnp.int32),
            pltpu.VMEM((_GATHER_W, _EMB), jnp.float32),
            pltpu.SemaphoreType.DMA,
        ],
        compiler_params=pltpu.CompilerParams(use_tc_tiling_on_sc=False),
    )
    return run(flat_table, idx_t)


def _out_body(e_ref, wo_ref, bo_ref, y_ref):
    e = e_ref[...]
    wo = wo_ref[...]
    acc = jnp.dot(e[0], wo[0 * _EMB:1 * _EMB, :],
                  preferred_element_type=jnp.float32)
    for h in range(1, _HEADS):
        acc = acc + jnp.dot(e[h], wo[h * _EMB:(h + 1) * _EMB, :],
                            preferred_element_type=jnp.float32)
    y_ref[...] = acc + bo_ref[...]


def _out_proj(rows3, Wo, bo):
    grid = _N // _BLK
    return pl.pallas_call(
        _out_body,
        grid=(grid,),
        in_specs=[
            pl.BlockSpec((_HEADS, _BLK, _EMB), lambda i: (_Z(), i, _Z())),
            pl.BlockSpec((_HEADS * _EMB, _STATE), lambda i: (_Z(), _Z())),
            pl.BlockSpec((1, _STATE), lambda i: (_Z(), _Z())),
        ],
        out_specs=pl.BlockSpec((_BLK, _STATE), lambda i: (i, _Z())),
        out_shape=jax.ShapeDtypeStruct((_N, _STATE), jnp.float32),
    )(rows3, Wo, bo.reshape(1, -1))


def kernel(scan_state, chars, Wp, bp, tables, Wo, bo):
    del chars
    x = scan_state.reshape(_N, _STATE)
    idx_t = _compute_keys(x, Wp, bp)
    flat_table = tables.reshape(_HEADS * _BUCKETS, _EMB)
    rows3 = _gather(flat_table, idx_t)
    y = _out_proj(rows3, Wo, bo)
    return y.reshape(_B, _T, _STATE)
